# baseline (speedup 1.0000x reference)
import functools

import jax
import jax.numpy as jnp
from jax import lax
from jax.experimental import pallas as pl
from jax.experimental.pallas import tpu as pltpu
from jax.experimental.pallas import tpu_sc as plsc

_BATCH = 16384
_NUM_SLOTS = 26
_DIM = 64
_B_FLAT = _BATCH * _NUM_SLOTS

_NC = 2
_NS = 16
_NW = _NC * _NS

_CHUNK = 128
_B_PER_W = _B_FLAT // _NW
_CHUNKS_PER_W = _B_PER_W // _CHUNK
_TOTAL_CHUNKS = _B_FLAT // _CHUNK
_NBUF = 8
_LOOKAHEAD = 6

_mesh = plsc.VectorSubcoreMesh(core_axis_name="c", subcore_axis_name="s")


@functools.partial(
    pl.kernel,
    mesh=_mesh,
    out_type=jax.ShapeDtypeStruct((_B_FLAT, _DIM), jnp.float32),
    compiler_params=pltpu.CompilerParams(use_tc_tiling_on_sc=False),
    scratch_types=[
        pltpu.VMEM((_CHUNKS_PER_W, _CHUNK), jnp.int32),
        pltpu.VMEM((_NBUF, _CHUNK, _DIM), jnp.float32),
    ] + [pltpu.SemaphoreType.DMA] * (2 * _NBUF),
)
def _sc_gather(idx_hbm, table_hbm, out_hbm, idx_v, rows_v, *sems):
    wid = lax.axis_index("s") * _NC + lax.axis_index("c")
    chunk0 = wid * _CHUNKS_PER_W
    base = wid * _B_PER_W

    pltpu.sync_copy(idx_hbm.at[pl.ds(chunk0, _CHUNKS_PER_W)], idx_v)

    gsems = sems[:_NBUF]
    wsems = sems[_NBUF:]

    def gather_start(c, s):
        pltpu.async_copy(table_hbm.at[idx_v.at[c]], rows_v.at[s], gsems[s])

    def gather_wait(c, s):
        pltpu.make_async_copy(table_hbm.at[idx_v.at[c]], rows_v.at[s],
                              gsems[s]).wait()

    def write_start(c, s):
        pltpu.async_copy(rows_v.at[s],
                         out_hbm.at[pl.ds(base + c * _CHUNK, _CHUNK)],
                         wsems[s])

    def write_wait(c, s):
        pltpu.make_async_copy(rows_v.at[s],
                              out_hbm.at[pl.ds(base + c * _CHUNK, _CHUNK)],
                              wsems[s]).wait()

    for s in range(_LOOKAHEAD):
        gather_start(s, s)

    def body(p, carry):
        for s in range(_NBUF):
            c = p * _NBUF + s
            f = c + _LOOKAHEAD
            sf = (s + _LOOKAHEAD) % _NBUF

            @pl.when(f < _CHUNKS_PER_W)
            def _():
                @pl.when(f >= _NBUF)
                def _():
                    write_wait(f - _NBUF, sf)
                gather_start(f, sf)

            gather_wait(c, s)
            write_start(c, s)
        return carry

    lax.fori_loop(0, _CHUNKS_PER_W // _NBUF, body, 0)

    for s in range(_NBUF):
        c = _CHUNKS_PER_W - _NBUF + s
        write_wait(c, s)


def kernel(slot_idx, table):
    idx2d = slot_idx.astype(jnp.int32).reshape(_TOTAL_CHUNKS, _CHUNK)
    out = _sc_gather(idx2d, table)
    return out.reshape(_BATCH, _NUM_SLOTS, _DIM)


# speedup vs baseline: 1.0745x; 1.0745x over previous
import functools

import jax
import jax.numpy as jnp
from jax import lax
from jax.experimental import pallas as pl
from jax.experimental.pallas import tpu as pltpu
from jax.experimental.pallas import tpu_sc as plsc

_BATCH = 16384
_NUM_SLOTS = 26
_DIM = 64
_NUM_EMBED = 1000000
_B_FLAT = _BATCH * _NUM_SLOTS

_NC = 2
_NS = 16
_NW = _NC * _NS

_CHUNK = 128
_B_PER_W = _B_FLAT // _NW
_CHUNKS_PER_W = _B_PER_W // _CHUNK
_TOTAL_CHUNKS = _B_FLAT // _CHUNK
_NBUF = 8
_LOOKAHEAD = 6

_mesh = plsc.VectorSubcoreMesh(core_axis_name="c", subcore_axis_name="s")


@functools.partial(
    pl.kernel,
    mesh=_mesh,
    out_type=jax.ShapeDtypeStruct((_B_FLAT, _DIM), jnp.float32),
    compiler_params=pltpu.CompilerParams(use_tc_tiling_on_sc=False),
    scratch_types=[
        pltpu.VMEM((_CHUNKS_PER_W, _CHUNK), jnp.int32),
        pltpu.VMEM((_NBUF, _CHUNK, _DIM), jnp.float32),
    ] + [pltpu.SemaphoreType.DMA] * (2 * _NBUF),
)
def _sc_gather(idx_hbm, table_hbm, out_hbm, idx_v, rows_v, *sems):
    wid = lax.axis_index("s") * _NC + lax.axis_index("c")
    chunk0 = wid * _CHUNKS_PER_W
    base = wid * _B_PER_W

    pltpu.sync_copy(idx_hbm.at[pl.ds(chunk0, _CHUNKS_PER_W)], idx_v)

    gsems = sems[:_NBUF]
    wsems = sems[_NBUF:]

    def gather_start(c, s):
        pltpu.async_copy(table_hbm.at[idx_v.at[c]], rows_v.at[s], gsems[s])

    def gather_wait(c, s):
        pltpu.make_async_copy(table_hbm.at[idx_v.at[c]], rows_v.at[s],
                              gsems[s]).wait()

    def write_start(c, s):
        pltpu.async_copy(rows_v.at[s],
                         out_hbm.at[pl.ds(base + c * _CHUNK, _CHUNK)],
                         wsems[s])

    def write_wait(c, s):
        pltpu.make_async_copy(rows_v.at[s],
                              out_hbm.at[pl.ds(base + c * _CHUNK, _CHUNK)],
                              wsems[s]).wait()

    for s in range(_LOOKAHEAD):
        gather_start(s, s)

    def body(p, carry):
        for s in range(_NBUF):
            c = p * _NBUF + s
            f = c + _LOOKAHEAD
            sf = (s + _LOOKAHEAD) % _NBUF

            @pl.when(f < _CHUNKS_PER_W)
            def _():
                @pl.when(f >= _NBUF)
                def _():
                    write_wait(f - _NBUF, sf)
                gather_start(f, sf)

            gather_wait(c, s)
            write_start(c, s)
        return carry

    lax.fori_loop(0, _CHUNKS_PER_W // _NBUF, body, 0)

    for s in range(_NBUF):
        c = _CHUNKS_PER_W - _NBUF + s
        write_wait(c, s)


def kernel(slot_idx, table):
    idx2d = (slot_idx.astype(jnp.int32) * 2).reshape(_TOTAL_CHUNKS, _CHUNK)
    tpad = jnp.pad(table, ((0, 0), (0, _DIM)))
    table_lin = tpad.reshape(2 * _NUM_EMBED, _DIM)
    out = _sc_gather(idx2d, table_lin)
    return out.reshape(_BATCH, _NUM_SLOTS, _DIM)
